# baseline (speedup 1.0000x reference)
import functools

import jax
import jax.numpy as jnp
import numpy as np
from jax import lax
from jax.experimental import pallas as pl
from jax.experimental.pallas import tpu as pltpu
from jax.experimental.pallas import tpu_sc as plsc

NC = 2
NS = 16
L = 16
NW = NC * NS
CHUNK = 80


def _make_spmm(N, E, D, K=CHUNK, packed=False):
    EW = E // NW
    assert EW * NW == E and EW % K == 0 and K % 8 == 0 and K <= 128
    nchunk = EW // K
    NB = 4
    NE = 8
    ngroup = nchunk // NE
    assert nchunk - ngroup * NE >= 2
    rows_a = ((N + NS - 1) // NS + 7) // 8 * 8
    rows_b = N - rows_a * (NS - 1)
    assert rows_b > 0
    mesh = plsc.VectorSubcoreMesh(core_axis_name="c", subcore_axis_name="s")

    @functools.partial(
        pl.kernel,
        out_type=jax.ShapeDtypeStruct((NC, N, D), jnp.float32),
        mesh=mesh,
        compiler_params=pltpu.CompilerParams(needs_layout_passes=False,
                                             use_tc_tiling_on_sc=False),
        scratch_types=[
            [pltpu.VMEM((3, K), jnp.int32) for _ in range(NE)],
            ([pltpu.VMEM((K, D // 2), jnp.int32) for _ in range(NB)]
             if packed else
             [pltpu.VMEM((K, D), jnp.float32) for _ in range(NB)]),
            ([pltpu.VMEM((K, D), jnp.float32) for _ in range(2)]
             if packed else
             [pltpu.VMEM((L,), jnp.float32)]),
            pltpu.VMEM_SHARED((N, D), jnp.float32),
            [pltpu.SemaphoreType.DMA for _ in range(NE)],
            [pltpu.SemaphoreType.DMA for _ in range(NB)],
            [pltpu.SemaphoreType.DMA for _ in range(NB)],
        ],
    )
    def spmm(edata_hbm, table_hbm, zeros_hbm, out_hbm,
             eb, rows, frows, acc_sh, isem, gsem, ssem):
        c = lax.axis_index("c")
        s = lax.axis_index("s")
        wid = s * NC + c
        crow = wid * nchunk

        @pl.when(s < NS - 1)
        def _():
            base = pl.multiple_of(s * rows_a, 8)
            pltpu.sync_copy(zeros_hbm.at[pl.ds(base, rows_a)],
                            acc_sh.at[pl.ds(base, rows_a)])

        @pl.when(s == NS - 1)
        def _():
            pltpu.sync_copy(zeros_hbm.at[pl.ds((NS - 1) * rows_a, rows_b)],
                            acc_sh.at[pl.ds((NS - 1) * rows_a, rows_b)])

        plsc.subcore_barrier()

        two = jnp.full((L,), 2, jnp.int32)

        def weight(b, es):
            @plsc.parallel_loop(0, K, 1, unroll=4 if packed else 8)
            def _(e):
                wbits = plsc.load_gather(eb[es],
                                         [two, jnp.full((L,), e, jnp.int32)])
                wb = plsc.bitcast(wbits, jnp.float32)
                if packed:
                    for j in range(D // (2 * L)):
                        x = rows[b][e, pl.ds(j * L, L)]
                        lo = plsc.bitcast(x << 16, jnp.float32)
                        hi = plsc.bitcast(x & jnp.int32(-65536), jnp.float32)
                        frows[b % 2][e, pl.ds(2 * j * L, L)] = lo * wb
                        frows[b % 2][e, pl.ds((2 * j + 1) * L, L)] = hi * wb
                else:
                    for j in range(D // L):
                        sl = pl.ds(j * L, L)
                        rows[b][e, sl] = rows[b][e, sl] * wb

        def _sbuf(b):
            return frows[b % 2] if packed else rows[b]

        def issue_edata(chunk_id, es):
            pltpu.async_copy(edata_hbm.at[crow + chunk_id], eb[es], isem[es])

        def wait_edata(chunk_id, es):
            pltpu.make_async_copy(edata_hbm.at[crow + chunk_id], eb[es],
                                  isem[es]).wait()

        def issue_gather(es, b):
            pltpu.async_copy(table_hbm.at[eb[es].at[0]], rows[b], gsem[b])

        def wait_gather(es, b):
            pltpu.make_async_copy(table_hbm.at[eb[es].at[0]], rows[b],
                                  gsem[b]).wait()

        def issue_scatter(es, b):
            pltpu.async_copy(_sbuf(b), acc_sh.at[eb[es].at[1]],
                             ssem[b], add=True)

        def wait_scatter(es, b):
            pltpu.make_async_copy(_sbuf(b), acc_sh.at[eb[es].at[1]],
                                  ssem[b]).wait()

        def step(g, b, es, guard):
            s_nxt = (b + 2) % NB
            e_nxt = (es + 2) % NE

            def drain():
                wait_scatter((es + 6) % NE, s_nxt)

            def prefetch():
                wait_edata(g + 2, e_nxt)
                issue_gather(e_nxt, s_nxt)

            def load_ahead():
                issue_edata(g + 4, (es + 4) % NE)

            if guard:
                pl.when(g >= 2)(drain)
                pl.when(g + 2 <= nchunk - 1)(prefetch)
                wait_gather(es, b)
                weight(b, es)
                issue_scatter(es, b)
                pl.when(g + 4 <= nchunk - 1)(load_ahead)
            else:
                if g >= 2:
                    drain()
                if g + 2 <= nchunk - 1:
                    prefetch()
                wait_gather(es, b)
                weight(b, es)
                issue_scatter(es, b)
                if g + 4 <= nchunk - 1:
                    load_ahead()

        for g0 in range(4):
            issue_edata(g0, g0)
        for g0 in range(2):
            wait_edata(g0, g0)
            issue_gather(g0, g0)

        def group(i, carry):
            for b in range(NE):
                g = NE * i + b
                step(g, b % NB, b, guard=True)
            return carry

        lax.fori_loop(0, ngroup, group, 0)
        for g in range(ngroup * NE, nchunk):
            step(g, g % NB, g % NE, guard=False)
        wait_scatter((nchunk - 2) % NE, (nchunk - 2) % NB)
        wait_scatter((nchunk - 1) % NE, (nchunk - 1) % NB)
        plsc.subcore_barrier()

        @pl.when(s < NS - 1)
        def _():
            base = pl.multiple_of(s * rows_a, 8)
            pltpu.sync_copy(acc_sh.at[pl.ds(base, rows_a)],
                            out_hbm.at[c, pl.ds(base, rows_a)])

        @pl.when(s == NS - 1)
        def _():
            pltpu.sync_copy(acc_sh.at[pl.ds((NS - 1) * rows_a, rows_b)],
                            out_hbm.at[c, pl.ds((NS - 1) * rows_a, rows_b)])

    return spmm


def _pack_perm(D):
    P = np.zeros(D, np.int32)
    for blk in range(D // 32):
        base = blk * 32
        P[base:base + 16] = base + np.arange(0, 32, 2)
        P[base + 16:base + 32] = base + np.arange(1, 32, 2)
    return P


def _as_packed_i32(t_bf16):
    n, d = t_bf16.shape
    return jax.lax.bitcast_convert_type(
        t_bf16.reshape(n, d // 2, 2), jnp.int32)


def _matmul(x, W, bm=1000, out_dtype=jnp.float32):
    n, f = x.shape
    h = W.shape[1]
    grid = n // bm

    def body(x_ref, w_ref, o_ref):
        z = jnp.dot(x_ref[...], w_ref[...], preferred_element_type=jnp.float32)
        o_ref[...] = z.astype(out_dtype)

    return pl.pallas_call(
        body,
        grid=(grid,),
        in_specs=[pl.BlockSpec((bm, f), lambda i: (i, 0)),
                  pl.BlockSpec((f, h), lambda i: (0, 0))],
        out_specs=pl.BlockSpec((bm, h), lambda i: (i, 0)),
        out_shape=jax.ShapeDtypeStruct((n, h), out_dtype),
    )(x, W)


def _fuse_matmul(p, b, W, relu, bm=1000, out_dtype=jnp.float32):
    _, n, d = p.shape
    h = W.shape[1]
    grid = n // bm
    b2 = b.reshape(1, d)

    def body(p_ref, b_ref, w_ref, o_ref):
        z = p_ref[0] + p_ref[1] + b_ref[...]
        if relu:
            z = jnp.maximum(z, 0.0)
        z = jnp.dot(z, w_ref[...], preferred_element_type=jnp.float32)
        o_ref[...] = z.astype(out_dtype)

    return pl.pallas_call(
        body,
        grid=(grid,),
        in_specs=[pl.BlockSpec((2, bm, d), lambda i: (0, i, 0)),
                  pl.BlockSpec((1, d), lambda i: (0, 0)),
                  pl.BlockSpec((d, h), lambda i: (0, 0))],
        out_specs=pl.BlockSpec((bm, h), lambda i: (i, 0)),
        out_shape=jax.ShapeDtypeStruct((n, h), out_dtype),
    )(p, b2, W)


def _fuse_logsoftmax(p, b, bm=1000):
    _, n, d = p.shape
    grid = n // bm
    b2 = b.reshape(1, d)

    def body(p_ref, b_ref, o_ref):
        z = p_ref[0] + p_ref[1] + b_ref[...]
        z = z - jnp.max(z, axis=1, keepdims=True)
        o_ref[...] = z - jnp.log(jnp.sum(jnp.exp(z), axis=1, keepdims=True))

    return pl.pallas_call(
        body,
        grid=(grid,),
        in_specs=[pl.BlockSpec((2, bm, d), lambda i: (0, i, 0)),
                  pl.BlockSpec((1, d), lambda i: (0, 0))],
        out_specs=pl.BlockSpec((bm, d), lambda i: (i, 0)),
        out_shape=jax.ShapeDtypeStruct((n, d), jnp.float32),
    )(p, b2)


def kernel(x, edge_index, edge_weight, W1, b1, W2, b2, W3, b3):
    n, nfeat = x.shape
    e = edge_weight.shape[0]
    nhid = W1.shape[1]
    nclass = W3.shape[1]
    src = edge_index[0].reshape(-1, CHUNK)
    dst = edge_index[1].reshape(-1, CHUNK)
    wbits = jax.lax.bitcast_convert_type(edge_weight, jnp.int32)
    edata = jnp.stack([src, dst, wbits.reshape(-1, CHUNK)], axis=1)
    zeros_h = jnp.zeros((n, nhid), jnp.float32)
    zeros_c = jnp.zeros((n, nclass), jnp.float32)

    spmm_h = _make_spmm(n, e, nhid, packed=True)
    spmm_c = _make_spmm(n, e, nclass)

    P = _pack_perm(nhid)
    s1 = _matmul(x, W1, out_dtype=jnp.bfloat16)
    p1 = spmm_h(edata, _as_packed_i32(s1), zeros_h)
    s2 = _fuse_matmul(p1, b1[P], W2[P, :], relu=True, out_dtype=jnp.bfloat16)
    p2 = spmm_h(edata, _as_packed_i32(s2), zeros_h)
    s3 = _fuse_matmul(p2, b2[P], W3[P, :], relu=False)
    p3 = spmm_c(edata, s3, zeros_c)
    return _fuse_logsoftmax(p3, b3)


# speedup vs baseline: 1.0838x; 1.0838x over previous
import functools

import jax
import jax.numpy as jnp
from jax import lax
from jax.experimental import pallas as pl
from jax.experimental.pallas import tpu as pltpu
from jax.experimental.pallas import tpu_sc as plsc

NC = 2
NS = 16
L = 16
NW = NC * NS
CHUNK = 40


def _make_spmm(N, E, D, K=CHUNK):
    EW = E // NW
    assert EW * NW == E and EW % K == 0 and K % 8 == 0 and K <= 128
    nchunk = EW // K
    NB = 4
    ngroup = nchunk // NB
    assert nchunk - ngroup * NB >= 2
    rows_a = ((N + NS - 1) // NS + 7) // 8 * 8
    rows_b = N - rows_a * (NS - 1)
    assert rows_b > 0
    mesh = plsc.VectorSubcoreMesh(core_axis_name="c", subcore_axis_name="s")

    @functools.partial(
        pl.kernel,
        out_type=jax.ShapeDtypeStruct((NC, N, D), jnp.float32),
        mesh=mesh,
        compiler_params=pltpu.CompilerParams(needs_layout_passes=False,
                                             use_tc_tiling_on_sc=False),
        scratch_types=[
            pltpu.VMEM((nchunk, K), jnp.int32),
            pltpu.VMEM((nchunk, K), jnp.int32),
            pltpu.VMEM((EW,), jnp.float32),
            [pltpu.VMEM((K, D), jnp.float32) for _ in range(NB)],
            pltpu.VMEM_SHARED((N, D), jnp.float32),
            [pltpu.SemaphoreType.DMA for _ in range(NB)],
            [pltpu.SemaphoreType.DMA for _ in range(NB)],
        ],
    )
    def spmm(src_hbm, dst_hbm, w_hbm, table_hbm, zeros_hbm, out_hbm,
             src_v, dst_v, w_v, rows, acc_sh, gsem, ssem):
        c = lax.axis_index("c")
        s = lax.axis_index("s")
        wid = s * NC + c

        pltpu.sync_copy(src_hbm.at[pl.ds(wid * nchunk, nchunk)], src_v)
        pltpu.sync_copy(dst_hbm.at[pl.ds(wid * nchunk, nchunk)], dst_v)
        pltpu.sync_copy(w_hbm.at[pl.ds(pl.multiple_of(wid * EW, 8), EW)], w_v)

        @pl.when(s < NS - 1)
        def _():
            base = pl.multiple_of(s * rows_a, 8)
            pltpu.sync_copy(zeros_hbm.at[pl.ds(base, rows_a)],
                            acc_sh.at[pl.ds(base, rows_a)])

        @pl.when(s == NS - 1)
        def _():
            pltpu.sync_copy(zeros_hbm.at[pl.ds((NS - 1) * rows_a, rows_b)],
                            acc_sh.at[pl.ds((NS - 1) * rows_a, rows_b)])

        plsc.subcore_barrier()

        def weight(rows_ref, chunk_id):
            @plsc.parallel_loop(0, K, 1, unroll=8)
            def _(e):
                wb = plsc.load_gather(
                    w_v, [jnp.full((L,), chunk_id * K + e, jnp.int32)])
                for j in range(D // L):
                    sl = pl.ds(j * L, L)
                    rows_ref[e, sl] = rows_ref[e, sl] * wb

        def issue_gather(chunk_id, b):
            pltpu.async_copy(table_hbm.at[src_v.at[chunk_id]], rows[b],
                             gsem[b])

        def wait_gather(chunk_id, b):
            pltpu.make_async_copy(table_hbm.at[src_v.at[chunk_id]], rows[b],
                                  gsem[b]).wait()

        def issue_scatter(chunk_id, b):
            pltpu.async_copy(rows[b], acc_sh.at[dst_v.at[chunk_id]],
                             ssem[b], add=True)

        def wait_scatter(chunk_id, b):
            pltpu.make_async_copy(rows[b], acc_sh.at[dst_v.at[chunk_id]],
                                  ssem[b]).wait()

        issue_gather(0, 0)
        issue_gather(1, 1)

        def group(i, carry):
            for b in range(NB):
                g = NB * i + b
                nxt = (b + 2) % NB

                @pl.when(g >= 2)
                def _():
                    wait_scatter(g - 2, nxt)

                @pl.when(g + 2 <= nchunk - 1)
                def _():
                    issue_gather(g + 2, nxt)

                wait_gather(g, b)
                weight(rows[b], g)
                issue_scatter(g, b)
            return carry

        lax.fori_loop(0, ngroup, group, 0)
        for g in range(ngroup * NB, nchunk):
            b = g % NB
            wait_scatter(g - 2, (b + 2) % NB)
            wait_gather(g, b)
            weight(rows[b], g)
            issue_scatter(g, b)
        wait_scatter(nchunk - 2, (nchunk - 2) % NB)
        wait_scatter(nchunk - 1, (nchunk - 1) % NB)
        plsc.subcore_barrier()

        @pl.when(s < NS - 1)
        def _():
            base = pl.multiple_of(s * rows_a, 8)
            pltpu.sync_copy(acc_sh.at[pl.ds(base, rows_a)],
                            out_hbm.at[c, pl.ds(base, rows_a)])

        @pl.when(s == NS - 1)
        def _():
            pltpu.sync_copy(acc_sh.at[pl.ds((NS - 1) * rows_a, rows_b)],
                            out_hbm.at[c, pl.ds((NS - 1) * rows_a, rows_b)])

    return spmm


def _matmul(x, W, bm=1000):
    n, f = x.shape
    h = W.shape[1]
    grid = n // bm

    def body(x_ref, w_ref, o_ref):
        o_ref[...] = jnp.dot(x_ref[...], w_ref[...],
                             preferred_element_type=jnp.float32)

    return pl.pallas_call(
        body,
        grid=(grid,),
        in_specs=[pl.BlockSpec((bm, f), lambda i: (i, 0)),
                  pl.BlockSpec((f, h), lambda i: (0, 0))],
        out_specs=pl.BlockSpec((bm, h), lambda i: (i, 0)),
        out_shape=jax.ShapeDtypeStruct((n, h), jnp.float32),
    )(x, W)


def _fuse_matmul(p, b, W, relu, bm=1000):
    _, n, d = p.shape
    h = W.shape[1]
    grid = n // bm
    b2 = b.reshape(1, d)

    def body(p_ref, b_ref, w_ref, o_ref):
        z = p_ref[0] + p_ref[1] + b_ref[...]
        if relu:
            z = jnp.maximum(z, 0.0)
        o_ref[...] = jnp.dot(z, w_ref[...], preferred_element_type=jnp.float32)

    return pl.pallas_call(
        body,
        grid=(grid,),
        in_specs=[pl.BlockSpec((2, bm, d), lambda i: (0, i, 0)),
                  pl.BlockSpec((1, d), lambda i: (0, 0)),
                  pl.BlockSpec((d, h), lambda i: (0, 0))],
        out_specs=pl.BlockSpec((bm, h), lambda i: (i, 0)),
        out_shape=jax.ShapeDtypeStruct((n, h), jnp.float32),
    )(p, b2, W)


def _fuse_logsoftmax(p, b, bm=1000):
    _, n, d = p.shape
    grid = n // bm
    b2 = b.reshape(1, d)

    def body(p_ref, b_ref, o_ref):
        z = p_ref[0] + p_ref[1] + b_ref[...]
        z = z - jnp.max(z, axis=1, keepdims=True)
        o_ref[...] = z - jnp.log(jnp.sum(jnp.exp(z), axis=1, keepdims=True))

    return pl.pallas_call(
        body,
        grid=(grid,),
        in_specs=[pl.BlockSpec((2, bm, d), lambda i: (0, i, 0)),
                  pl.BlockSpec((1, d), lambda i: (0, 0))],
        out_specs=pl.BlockSpec((bm, d), lambda i: (i, 0)),
        out_shape=jax.ShapeDtypeStruct((n, d), jnp.float32),
    )(p, b2)


def kernel(x, edge_index, edge_weight, W1, b1, W2, b2, W3, b3):
    n, nfeat = x.shape
    e = edge_weight.shape[0]
    nhid = W1.shape[1]
    nclass = W3.shape[1]
    src = edge_index[0].reshape(-1, CHUNK)
    dst = edge_index[1].reshape(-1, CHUNK)
    zeros_h = jnp.zeros((n, nhid), jnp.float32)
    zeros_c = jnp.zeros((n, nclass), jnp.float32)

    spmm_h = _make_spmm(n, e, nhid)
    spmm_c = _make_spmm(n, e, nclass)

    s1 = _matmul(x, W1)
    p1 = spmm_h(src, dst, edge_weight, s1, zeros_h)
    s2 = _fuse_matmul(p1, b1, W2, relu=True)
    p2 = spmm_h(src, dst, edge_weight, s2, zeros_h)
    s3 = _fuse_matmul(p2, b2, W3, relu=False)
    p3 = spmm_c(src, dst, edge_weight, s3, zeros_c)
    return _fuse_logsoftmax(p3, b3)


# speedup vs baseline: 1.1177x; 1.0313x over previous
import functools

import jax
import jax.numpy as jnp
from jax import lax
from jax.experimental import pallas as pl
from jax.experimental.pallas import tpu as pltpu
from jax.experimental.pallas import tpu_sc as plsc

NC = 2
NS = 16
L = 16
NW = NC * NS
CHUNK = 40


def _make_spmm(N, E, D, K=CHUNK):
    EW = E // NW
    assert EW * NW == E and EW % K == 0 and K % 8 == 0 and K <= 128
    nchunk = EW // K
    NB = 4
    ngroup = nchunk // NB
    assert nchunk - ngroup * NB >= 2
    rows_a = -(-(N // NS) // K) * K
    rows_b = N - rows_a * (NS - 1)
    assert rows_b > 0 and rows_a % K == 0 and rows_b % K == 0
    mesh = plsc.VectorSubcoreMesh(core_axis_name="c", subcore_axis_name="s")

    @functools.partial(
        pl.kernel,
        out_type=jax.ShapeDtypeStruct((NC, N, D), jnp.float32),
        mesh=mesh,
        compiler_params=pltpu.CompilerParams(needs_layout_passes=False,
                                             use_tc_tiling_on_sc=False),
        scratch_types=[
            pltpu.VMEM((nchunk, K), jnp.int32),
            pltpu.VMEM((nchunk, K), jnp.int32),
            pltpu.VMEM((EW,), jnp.float32),
            [pltpu.VMEM((K, D), jnp.float32) for _ in range(NB)],
            pltpu.VMEM_SHARED((N, D), jnp.float32),
            [pltpu.SemaphoreType.DMA for _ in range(NB)],
            [pltpu.SemaphoreType.DMA for _ in range(NB)],
        ],
    )
    def spmm(src_hbm, dst_hbm, w_hbm, table_hbm, out_hbm,
             src_v, dst_v, w_v, rows, acc_sh, gsem, ssem):
        c = lax.axis_index("c")
        s = lax.axis_index("s")
        wid = s * NC + c

        pltpu.sync_copy(src_hbm.at[pl.ds(wid * nchunk, nchunk)], src_v)
        pltpu.sync_copy(dst_hbm.at[pl.ds(wid * nchunk, nchunk)], dst_v)
        pltpu.sync_copy(w_hbm.at[pl.ds(pl.multiple_of(wid * EW, 8), EW)], w_v)

        def weight(rows_ref, chunk_id):
            @plsc.parallel_loop(0, K, 1, unroll=8)
            def _(e):
                wb = plsc.load_gather(
                    w_v, [jnp.full((L,), chunk_id * K + e, jnp.int32)])
                for j in range(D // L):
                    sl = pl.ds(j * L, L)
                    rows_ref[e, sl] = rows_ref[e, sl] * wb

        def issue_gather(chunk_id, b):
            pltpu.async_copy(table_hbm.at[src_v.at[chunk_id]], rows[b],
                             gsem[b])

        def wait_gather(chunk_id, b):
            pltpu.make_async_copy(table_hbm.at[src_v.at[chunk_id]], rows[b],
                                  gsem[b]).wait()

        def issue_scatter(chunk_id, b):
            pltpu.async_copy(rows[b], acc_sh.at[dst_v.at[chunk_id]],
                             ssem[b], add=True)

        def wait_scatter(chunk_id, b):
            pltpu.make_async_copy(rows[b], acc_sh.at[dst_v.at[chunk_id]],
                                  ssem[b]).wait()

        issue_gather(0, 0)
        issue_gather(1, 1)

        @plsc.parallel_loop(0, K, 1)
        def _(e):
            for j in range(D // L):
                rows[2][e, pl.ds(j * L, L)] = jnp.zeros((L,), jnp.float32)

        @pl.when(s < NS - 1)
        def _():
            base = pl.multiple_of(s * rows_a, 8)
            for i in range(rows_a // K):
                pltpu.sync_copy(rows[2], acc_sh.at[pl.ds(base + i * K, K)])

        @pl.when(s == NS - 1)
        def _():
            for i in range(rows_b // K):
                pltpu.sync_copy(
                    rows[2],
                    acc_sh.at[pl.ds((NS - 1) * rows_a + i * K, K)])

        plsc.subcore_barrier()

        def group(i, carry):
            for b in range(NB):
                g = NB * i + b
                nxt = (b + 2) % NB

                @pl.when(g >= 2)
                def _():
                    wait_scatter(g - 2, nxt)

                @pl.when(g + 2 <= nchunk - 1)
                def _():
                    issue_gather(g + 2, nxt)

                wait_gather(g, b)
                weight(rows[b], g)
                issue_scatter(g, b)
            return carry

        lax.fori_loop(0, ngroup, group, 0)
        for g in range(ngroup * NB, nchunk):
            b = g % NB
            wait_scatter(g - 2, (b + 2) % NB)
            wait_gather(g, b)
            weight(rows[b], g)
            issue_scatter(g, b)
        wait_scatter(nchunk - 2, (nchunk - 2) % NB)
        wait_scatter(nchunk - 1, (nchunk - 1) % NB)
        plsc.subcore_barrier()

        @pl.when(s < NS - 1)
        def _():
            base = pl.multiple_of(s * rows_a, 8)
            pltpu.sync_copy(acc_sh.at[pl.ds(base, rows_a)],
                            out_hbm.at[c, pl.ds(base, rows_a)])

        @pl.when(s == NS - 1)
        def _():
            pltpu.sync_copy(acc_sh.at[pl.ds((NS - 1) * rows_a, rows_b)],
                            out_hbm.at[c, pl.ds((NS - 1) * rows_a, rows_b)])

    return spmm


def _matmul(x, W, bm=1000):
    n, f = x.shape
    h = W.shape[1]
    grid = n // bm

    def body(x_ref, w_ref, o_ref):
        o_ref[...] = jnp.dot(x_ref[...], w_ref[...],
                             preferred_element_type=jnp.float32)

    return pl.pallas_call(
        body,
        grid=(grid,),
        in_specs=[pl.BlockSpec((bm, f), lambda i: (i, 0)),
                  pl.BlockSpec((f, h), lambda i: (0, 0))],
        out_specs=pl.BlockSpec((bm, h), lambda i: (i, 0)),
        out_shape=jax.ShapeDtypeStruct((n, h), jnp.float32),
    )(x, W)


def _fuse_matmul(p, b, W, relu, bm=1000):
    _, n, d = p.shape
    h = W.shape[1]
    grid = n // bm
    b2 = b.reshape(1, d)

    def body(p_ref, b_ref, w_ref, o_ref):
        z = p_ref[0] + p_ref[1] + b_ref[...]
        if relu:
            z = jnp.maximum(z, 0.0)
        o_ref[...] = jnp.dot(z, w_ref[...], preferred_element_type=jnp.float32)

    return pl.pallas_call(
        body,
        grid=(grid,),
        in_specs=[pl.BlockSpec((2, bm, d), lambda i: (0, i, 0)),
                  pl.BlockSpec((1, d), lambda i: (0, 0)),
                  pl.BlockSpec((d, h), lambda i: (0, 0))],
        out_specs=pl.BlockSpec((bm, h), lambda i: (i, 0)),
        out_shape=jax.ShapeDtypeStruct((n, h), jnp.float32),
    )(p, b2, W)


def _fuse_logsoftmax(p, b, bm=1000):
    _, n, d = p.shape
    grid = n // bm
    b2 = b.reshape(1, d)

    def body(p_ref, b_ref, o_ref):
        z = p_ref[0] + p_ref[1] + b_ref[...]
        z = z - jnp.max(z, axis=1, keepdims=True)
        o_ref[...] = z - jnp.log(jnp.sum(jnp.exp(z), axis=1, keepdims=True))

    return pl.pallas_call(
        body,
        grid=(grid,),
        in_specs=[pl.BlockSpec((2, bm, d), lambda i: (0, i, 0)),
                  pl.BlockSpec((1, d), lambda i: (0, 0))],
        out_specs=pl.BlockSpec((bm, d), lambda i: (i, 0)),
        out_shape=jax.ShapeDtypeStruct((n, d), jnp.float32),
    )(p, b2)


def kernel(x, edge_index, edge_weight, W1, b1, W2, b2, W3, b3):
    n, nfeat = x.shape
    e = edge_weight.shape[0]
    nhid = W1.shape[1]
    nclass = W3.shape[1]
    src = edge_index[0].reshape(-1, CHUNK)
    dst = edge_index[1].reshape(-1, CHUNK)

    spmm_h = _make_spmm(n, e, nhid)
    spmm_c = _make_spmm(n, e, nclass)

    s1 = _matmul(x, W1)
    p1 = spmm_h(src, dst, edge_weight, s1)
    s2 = _fuse_matmul(p1, b1, W2, relu=True)
    p2 = spmm_h(src, dst, edge_weight, s2)
    s3 = _fuse_matmul(p2, b2, W3, relu=False)
    p3 = spmm_c(src, dst, edge_weight, s3)
    return _fuse_logsoftmax(p3, b3)
